# baseline (speedup 1.0000x reference)
import functools

import jax
import jax.numpy as jnp
from jax import lax
from jax.experimental import pallas as pl
from jax.experimental.pallas import tpu as pltpu
from jax.experimental.pallas import tpu_sc as plsc

NC = 2
NS = 16
CHUNK = 128


def _sc_aggregate(n_rows, nch, dh):
    rpt = n_rows // NS
    nsup = nch // 2

    mesh = plsc.VectorSubcoreMesh(core_axis_name="c", subcore_axis_name="s")

    @functools.partial(
        pl.kernel,
        out_type=(
            jax.ShapeDtypeStruct((n_rows, 2 * dh), jnp.float32),
            jax.ShapeDtypeStruct((NC, n_rows, 16), jnp.float32),
        ),
        mesh=mesh,
        scratch_types=[
            pltpu.VMEM((nch, CHUNK), jnp.int32),
            pltpu.VMEM((nch, CHUNK), jnp.int32),
            pltpu.VMEM((CHUNK, dh), jnp.float32),
            pltpu.VMEM((CHUNK, dh), jnp.float32),
            pltpu.VMEM((CHUNK, 16), jnp.float32),
            pltpu.VMEM_SHARED((n_rows, dh), jnp.float32),
            pltpu.VMEM_SHARED((n_rows, 16), jnp.float32),
            pltpu.SemaphoreType.DMA,
            pltpu.SemaphoreType.DMA,
        ],
        compiler_params=pltpu.CompilerParams(use_tc_tiling_on_sc=False),
    )
    def sc_kernel(x_hbm, src_hbm, dst_hbm,
                  agg_out, deg_out,
                  src_v, dst_v, buf0, buf1, ones_v, agg_sh, deg_sh,
                  gsem0, gsem1):
        cid = lax.axis_index("c")
        sid = lax.axis_index("s")
        wid = cid * NS + sid
        base = sid * rpt

        pltpu.sync_copy(src_hbm.at[wid], src_v)
        pltpu.sync_copy(dst_hbm.at[sid], dst_v)

        zv = jnp.zeros((16,), jnp.float32)

        def zrow(r, carry):
            for gc in range(dh // 16):
                buf0[r, pl.ds(16 * gc, 16)] = zv
            ones_v[r, pl.ds(0, 16)] = zv
            return carry

        lax.fori_loop(0, CHUNK, zrow, 0, unroll=False)

        full = rpt // CHUNK
        tail = rpt - full * CHUNK
        for k in range(full):
            pltpu.sync_copy(buf0, agg_sh.at[pl.ds(base + k * CHUNK, CHUNK)])
            pltpu.sync_copy(ones_v, deg_sh.at[pl.ds(base + k * CHUNK, CHUNK)])
        if tail:
            pltpu.sync_copy(buf0.at[pl.ds(0, tail)],
                            agg_sh.at[pl.ds(base + full * CHUNK, tail)])
            pltpu.sync_copy(ones_v.at[pl.ds(0, tail)],
                            deg_sh.at[pl.ds(base + full * CHUNK, tail)])

        ov = jnp.ones((16,), jnp.float32)

        def orow(r, carry):
            ones_v[r, pl.ds(0, 16)] = ov
            return carry

        lax.fori_loop(0, CHUNK, orow, 0, unroll=False)
        plsc.subcore_barrier()

        pltpu.async_copy(x_hbm.at[src_v.at[0]], buf0, gsem0)

        def body(g, carry):
            pltpu.async_copy(x_hbm.at[src_v.at[g + 1]], buf1, gsem1)
            pltpu.make_async_copy(x_hbm.at[src_v.at[g]], buf0, gsem0).wait()
            pltpu.sync_copy(buf0, agg_sh.at[dst_v.at[g]], add=True)

            @pl.when(cid == 0)
            def _():
                pltpu.sync_copy(ones_v, deg_sh.at[dst_v.at[g]], add=True)

            @pl.when(g + 2 < nch)
            def _():
                pltpu.async_copy(x_hbm.at[src_v.at[g + 2]], buf0, gsem0)

            pltpu.make_async_copy(x_hbm.at[src_v.at[g + 1]], buf1, gsem1).wait()
            pltpu.sync_copy(buf1, agg_sh.at[dst_v.at[g + 1]], add=True)

            @pl.when(cid == 1)
            def _():
                pltpu.sync_copy(ones_v, deg_sh.at[dst_v.at[g + 1]], add=True)

            return carry

        lax.fori_loop(0, nsup, lambda i, c: body(2 * i, c), 0, unroll=False)

        plsc.subcore_barrier()

        @pl.when(cid == 0)
        def _():
            pltpu.sync_copy(agg_sh.at[pl.ds(base, rpt)],
                            agg_out.at[pl.ds(base, rpt), pl.ds(0, dh)])

        @pl.when(cid == 1)
        def _():
            pltpu.sync_copy(agg_sh.at[pl.ds(base, rpt)],
                            agg_out.at[pl.ds(base, rpt), pl.ds(dh, dh)])

        pltpu.sync_copy(deg_sh.at[pl.ds(base, rpt)],
                        deg_out.at[cid, pl.ds(base, rpt)])

    return sc_kernel


def _tc_transform(n_nodes, d_feat, block_rows):
    def body(x_ref, p_ref, dp_ref, wn_ref, ws_ref, b_ref, out_ref):
        deg = dp_ref[0, :, 0:1] + dp_ref[1, :, 0:1]
        agg = p_ref[...] / jnp.maximum(deg, 1.0)
        acc = jnp.dot(agg, wn_ref[...], preferred_element_type=jnp.float32)
        acc += jnp.dot(x_ref[...], ws_ref[...], preferred_element_type=jnp.float32)
        out_ref[...] = jnp.maximum(acc + b_ref[...], 0.0)

    grid = n_nodes // block_rows
    return pl.pallas_call(
        body,
        grid=(grid,),
        in_specs=[
            pl.BlockSpec((block_rows, d_feat), lambda i: (i, 0)),
            pl.BlockSpec((block_rows, d_feat), lambda i: (i, 0)),
            pl.BlockSpec((NC, block_rows, 16), lambda i: (0, i, 0)),
            pl.BlockSpec((d_feat, d_feat), lambda i: (0, 0)),
            pl.BlockSpec((d_feat, d_feat), lambda i: (0, 0)),
            pl.BlockSpec((1, d_feat), lambda i: (0, 0)),
        ],
        out_specs=pl.BlockSpec((block_rows, d_feat), lambda i: (i, 0)),
        out_shape=jax.ShapeDtypeStruct((n_nodes, d_feat), jnp.float32),
        compiler_params=pltpu.CompilerParams(
            dimension_semantics=("arbitrary",),
        ),
    )


def kernel(x, edge_index, W_self, W_neigh, b):
    n, d = x.shape
    dh = d // 2
    e = edge_index.shape[1]

    nch = -(-e // (NS * CHUNK))
    nch += nch % 2
    e_pad = NS * CHUNK * nch
    ei = jnp.pad(edge_index, ((0, 0), (0, e_pad - e)), constant_values=n)
    src = jnp.stack([ei[0], ei[0] + n]).reshape(NC * NS, nch, CHUNK)
    dst = ei[1].reshape(NS, nch, CHUNK)

    xcat = jnp.concatenate(
        [x[:, :dh], x[:, dh:], jnp.zeros((8, dh), jnp.float32)], axis=0)

    rpt = 8 * (-(-(n + 1) // (NS * 8)))
    n_rows = NS * rpt

    agg, deg_p = _sc_aggregate(n_rows, nch, dh)(xcat, src, dst)

    block_rows = 1000 if n % 1000 == 0 else 8
    out = _tc_transform(n, d, block_rows)(
        x, agg, deg_p, W_neigh, W_self, b.reshape(1, d))
    return out


# speedup vs baseline: 1.0322x; 1.0322x over previous
import functools

import jax
import jax.numpy as jnp
from jax import lax
from jax.experimental import pallas as pl
from jax.experimental.pallas import tpu as pltpu
from jax.experimental.pallas import tpu_sc as plsc

NC = 2
NS = 16
CHUNK = 128


def _sc_aggregate(n_rows, nch, dh):
    rpt = n_rows // NS
    nsup = nch // 2

    mesh = plsc.VectorSubcoreMesh(core_axis_name="c", subcore_axis_name="s")

    @functools.partial(
        pl.kernel,
        out_type=(
            jax.ShapeDtypeStruct((n_rows, 2 * dh), jnp.float32),
            jax.ShapeDtypeStruct((NC, n_rows, 16), jnp.float32),
        ),
        mesh=mesh,
        scratch_types=[
            pltpu.VMEM((nch, CHUNK), jnp.int32),
            pltpu.VMEM((nch, CHUNK), jnp.int32),
            pltpu.VMEM((CHUNK, dh), jnp.float32),
            pltpu.VMEM((CHUNK, dh), jnp.float32),
            pltpu.VMEM((CHUNK, 16), jnp.float32),
            pltpu.VMEM_SHARED((n_rows, dh), jnp.float32),
            pltpu.VMEM_SHARED((n_rows, 16), jnp.float32),
            pltpu.SemaphoreType.DMA,
            pltpu.SemaphoreType.DMA,
        ],
        compiler_params=pltpu.CompilerParams(use_tc_tiling_on_sc=False),
    )
    def sc_kernel(x_hbm, src_hbm, dst_hbm,
                  agg_out, deg_out,
                  src_v, dst_v, buf0, buf1, ones_v, agg_sh, deg_sh,
                  gsem0, gsem1):
        cid = lax.axis_index("c")
        sid = lax.axis_index("s")
        wid = cid * NS + sid
        base = sid * rpt

        pltpu.sync_copy(src_hbm.at[wid], src_v)
        pltpu.sync_copy(dst_hbm.at[sid], dst_v)

        zv = jnp.zeros((16,), jnp.float32)

        def zrow(r, carry):
            for gc in range(dh // 16):
                buf0[r, pl.ds(16 * gc, 16)] = zv
            ones_v[r, pl.ds(0, 16)] = zv
            return carry

        lax.fori_loop(0, CHUNK, zrow, 0, unroll=False)

        full = rpt // CHUNK
        tail = rpt - full * CHUNK
        for k in range(full):
            pltpu.sync_copy(buf0, agg_sh.at[pl.ds(base + k * CHUNK, CHUNK)])
            pltpu.sync_copy(ones_v, deg_sh.at[pl.ds(base + k * CHUNK, CHUNK)])
        if tail:
            pltpu.sync_copy(buf0.at[pl.ds(0, tail)],
                            agg_sh.at[pl.ds(base + full * CHUNK, tail)])
            pltpu.sync_copy(ones_v.at[pl.ds(0, tail)],
                            deg_sh.at[pl.ds(base + full * CHUNK, tail)])

        ov = jnp.ones((16,), jnp.float32)

        def orow(r, carry):
            ones_v[r, pl.ds(0, 16)] = ov
            return carry

        lax.fori_loop(0, CHUNK, orow, 0, unroll=False)
        plsc.subcore_barrier()

        pltpu.async_copy(x_hbm.at[src_v.at[0]], buf0, gsem0)

        def body(g, carry):
            pltpu.async_copy(x_hbm.at[src_v.at[g + 1]], buf1, gsem1)
            pltpu.make_async_copy(x_hbm.at[src_v.at[g]], buf0, gsem0).wait()
            pltpu.sync_copy(buf0, agg_sh.at[dst_v.at[g]], add=True)

            @pl.when(cid == 0)
            def _():
                pltpu.sync_copy(ones_v, deg_sh.at[dst_v.at[g]], add=True)

            @pl.when(g + 2 < nch)
            def _():
                pltpu.async_copy(x_hbm.at[src_v.at[g + 2]], buf0, gsem0)

            pltpu.make_async_copy(x_hbm.at[src_v.at[g + 1]], buf1, gsem1).wait()
            pltpu.sync_copy(buf1, agg_sh.at[dst_v.at[g + 1]], add=True)

            @pl.when(cid == 1)
            def _():
                pltpu.sync_copy(ones_v, deg_sh.at[dst_v.at[g + 1]], add=True)

            return carry

        lax.fori_loop(0, nsup, lambda i, c: body(2 * i, c), 0, unroll=False)

        plsc.subcore_barrier()

        @pl.when(cid == 0)
        def _():
            pltpu.sync_copy(agg_sh.at[pl.ds(base, rpt)],
                            agg_out.at[pl.ds(base, rpt), pl.ds(0, dh)])

        @pl.when(cid == 1)
        def _():
            pltpu.sync_copy(agg_sh.at[pl.ds(base, rpt)],
                            agg_out.at[pl.ds(base, rpt), pl.ds(dh, dh)])

        pltpu.sync_copy(deg_sh.at[pl.ds(base, rpt)],
                        deg_out.at[cid, pl.ds(base, rpt)])

    return sc_kernel


def _tc_transform(n_nodes, d_feat, block_rows):
    def body(x_ref, p_ref, dp_ref, wn_ref, ws_ref, b_ref, out_ref):
        deg = dp_ref[0, :, 0:1] + dp_ref[1, :, 0:1]
        agg = p_ref[...] / jnp.maximum(deg, 1.0)
        acc = jnp.dot(agg, wn_ref[...], preferred_element_type=jnp.float32)
        acc += jnp.dot(x_ref[...], ws_ref[...], preferred_element_type=jnp.float32)
        out_ref[...] = jnp.maximum(acc + b_ref[...], 0.0)

    grid = n_nodes // block_rows
    return pl.pallas_call(
        body,
        grid=(grid,),
        in_specs=[
            pl.BlockSpec((block_rows, d_feat), lambda i: (i, 0)),
            pl.BlockSpec((block_rows, d_feat), lambda i: (i, 0)),
            pl.BlockSpec((NC, block_rows, 16), lambda i: (0, i, 0)),
            pl.BlockSpec((d_feat, d_feat), lambda i: (0, 0)),
            pl.BlockSpec((d_feat, d_feat), lambda i: (0, 0)),
            pl.BlockSpec((1, d_feat), lambda i: (0, 0)),
        ],
        out_specs=pl.BlockSpec((block_rows, d_feat), lambda i: (i, 0)),
        out_shape=jax.ShapeDtypeStruct((n_nodes, d_feat), jnp.float32),
        compiler_params=pltpu.CompilerParams(
            dimension_semantics=("arbitrary",),
        ),
    )


def kernel(x, edge_index, W_self, W_neigh, b):
    n, d = x.shape
    dh = d // 2
    e = edge_index.shape[1]

    nch = -(-e // (NS * CHUNK))
    nch += nch % 2
    e_pad = NS * CHUNK * nch
    src = jnp.concatenate(
        [edge_index[0], jnp.zeros((e_pad - e,), jnp.int32)]).reshape(NS, nch, CHUNK)
    src = jnp.stack([src, src + n]).reshape(NC * NS, nch, CHUNK)
    dst = jnp.concatenate(
        [edge_index[1], jnp.full((e_pad - e,), n, jnp.int32)]).reshape(NS, nch, CHUNK)

    xcat = jnp.concatenate([x[:, :dh], x[:, dh:]], axis=0)

    rpt = 8 * (-(-(n + 1) // (NS * 8)))
    n_rows = NS * rpt

    agg, deg_p = _sc_aggregate(n_rows, nch, dh)(xcat, src, dst)

    block_rows = 1000 if n % 1000 == 0 else 8
    out = _tc_transform(n, d, block_rows)(
        x, agg, deg_p, W_neigh, W_self, b.reshape(1, d))
    return out
